# Initial kernel scaffold; baseline (speedup 1.0000x reference)
import jax
import jax.numpy as jnp
from jax.experimental import pallas as pl


def kernel(x, W_pi, b_pi, W_sigma, b_sigma, W_mu, b_mu):
    raise NotImplementedError("write your pallas kernel here")


# speedup vs baseline: 3.0288x; 3.0288x over previous
import functools

import jax
import jax.numpy as jnp
from jax.experimental import pallas as pl
from jax.experimental.pallas import tpu as pltpu

_B, _T, _D_IN, _G, _D = 8, 2048, 512, 8, 256
_N = _B * _T
_BLK = 512


def _fused_body(x_ref, wpi_ref, bpi_ref, wsig_ref, bsig_ref, wmu_ref, bmu_ref,
                mu_ref, sig_ref):
    x = x_ref[...]
    logits = jnp.dot(x, wpi_ref[...], preferred_element_type=jnp.float32)
    logits = logits + bpi_ref[...]
    g = jnp.argmax(logits, axis=1).astype(jnp.int32)

    lane_group = jax.lax.broadcasted_iota(jnp.int32, (_BLK, _G * _D), 1) // _D
    keep = lane_group == g[:, None]

    mu_full = jnp.dot(x, wmu_ref[...], preferred_element_type=jnp.float32)
    mu_full = jnp.where(keep, mu_full + bmu_ref[...], 0.0)
    acc_mu = jnp.zeros((_BLK, _D), jnp.float32)
    for k in range(_G):
        acc_mu = acc_mu + mu_full[:, k * _D:(k + 1) * _D]
    mu_ref[...] = acc_mu

    sig_full = jnp.dot(x, wsig_ref[...], preferred_element_type=jnp.float32)
    sig_full = jnp.where(keep, sig_full + bsig_ref[...], 0.0)
    acc_sig = jnp.zeros((_BLK, _D), jnp.float32)
    for k in range(_G):
        acc_sig = acc_sig + sig_full[:, k * _D:(k + 1) * _D]
    sig_ref[...] = jnp.exp(acc_sig)


@jax.jit
def kernel(x, W_pi, b_pi, W_sigma, b_sigma, W_mu, b_mu):
    xf = x.reshape(_N, _D_IN)
    grid = (_N // _BLK,)
    full = lambda i: (0, 0)
    mu, sig = pl.pallas_call(
        _fused_body,
        grid=grid,
        in_specs=[
            pl.BlockSpec((_BLK, _D_IN), lambda i: (i, 0)),
            pl.BlockSpec((_D_IN, _G), full),
            pl.BlockSpec((_G,), lambda i: (0,)),
            pl.BlockSpec((_D_IN, _G * _D), full),
            pl.BlockSpec((_G * _D,), lambda i: (0,)),
            pl.BlockSpec((_D_IN, _G * _D), full),
            pl.BlockSpec((_G * _D,), lambda i: (0,)),
        ],
        out_specs=[
            pl.BlockSpec((_BLK, _D), lambda i: (i, 0)),
            pl.BlockSpec((_BLK, _D), lambda i: (i, 0)),
        ],
        out_shape=[
            jax.ShapeDtypeStruct((_N, _D), jnp.float32),
            jax.ShapeDtypeStruct((_N, _D), jnp.float32),
        ],
        compiler_params=pltpu.CompilerParams(
            dimension_semantics=("arbitrary",),
        ),
    )(xf, W_pi, b_pi, W_sigma, b_sigma, W_mu, b_mu)
    return mu.reshape(_B, _T, _D), sig.reshape(_B, _T, _D)
